# baseline (speedup 1.0000x reference)
import jax
import jax.numpy as jnp
from jax.experimental import pallas as pl
from jax.experimental.pallas import tpu as pltpu

_TILE = 256


def _vq_step(z_ref, m_ref, cb_ref, q_ref, idx_ref, sumsq_ref, cnt_ref):
    i = pl.program_id(0)
    z = z_ref[0]
    cb = cb_ref[...]
    k = cb.shape[0]

    z2 = jnp.sum(z * z, axis=1, keepdims=True)
    c2 = jnp.sum(cb * cb, axis=1)
    dots = jax.lax.dot_general(
        z, cb, (((1,), (1,)), ((), ())),
        preferred_element_type=jnp.float32)
    dist = (z2 - 2.0 * dots) + c2[None, :]

    minv = jnp.min(dist, axis=1, keepdims=True)
    kiota = jax.lax.broadcasted_iota(jnp.int32, dist.shape, 1)
    idx = jnp.min(jnp.where(dist == minv, kiota, k),
                  axis=1, keepdims=True)
    idx_ref[0] = idx

    onehot = (kiota == idx).astype(jnp.float32)
    q = jnp.dot(onehot, cb, preferred_element_type=jnp.float32,
                precision=jax.lax.Precision.HIGHEST)
    q_ref[0] = q

    m = m_ref[0]
    diff = z - q
    psum = jnp.sum(diff * diff * m, keepdims=True)
    pcnt = jnp.sum(m, keepdims=True)

    @pl.when(i == 0)
    def _init():
        sumsq_ref[...] = jnp.zeros((1, 1), jnp.float32)
        cnt_ref[...] = jnp.zeros((1, 1), jnp.float32)

    sumsq_ref[...] += psum
    cnt_ref[...] += pcnt


def kernel(z, mask, codebook):
    b, t, d = z.shape
    k = codebook.shape[0]
    n = b * t
    nt = n // _TILE

    z3 = z.reshape(nt, _TILE, d)
    m3 = mask.astype(jnp.float32).reshape(nt, _TILE, 1)

    grid = (nt,)
    q3, idx3, sumsq, cnt = pl.pallas_call(
        _vq_step,
        grid=grid,
        in_specs=[
            pl.BlockSpec((1, _TILE, d), lambda i: (i, 0, 0)),
            pl.BlockSpec((1, _TILE, 1), lambda i: (i, 0, 0)),
            pl.BlockSpec((k, d), lambda i: (0, 0)),
        ],
        out_specs=[
            pl.BlockSpec((1, _TILE, d), lambda i: (i, 0, 0)),
            pl.BlockSpec((1, _TILE, 1), lambda i: (i, 0, 0)),
            pl.BlockSpec((1, 1), lambda i: (0, 0)),
            pl.BlockSpec((1, 1), lambda i: (0, 0)),
        ],
        out_shape=[
            jax.ShapeDtypeStruct((nt, _TILE, d), jnp.float32),
            jax.ShapeDtypeStruct((nt, _TILE, 1), jnp.int32),
            jax.ShapeDtypeStruct((1, 1), jnp.float32),
            jax.ShapeDtypeStruct((1, 1), jnp.float32),
        ],
        compiler_params=pltpu.CompilerParams(
            dimension_semantics=("arbitrary",),
        ),
    )(z3, m3, codebook)

    quantized = q3.reshape(b, t, d)
    indices = idx3.reshape(b, t)
    denom = jnp.maximum(cnt[0, 0], 1.0) * jnp.float32(d)
    sum_commit_loss = sumsq[0, 0] / denom
    return quantized, indices, sum_commit_loss


# speedup vs baseline: 1.3799x; 1.3799x over previous
import jax
import jax.numpy as jnp
from jax.experimental import pallas as pl
from jax.experimental.pallas import tpu as pltpu

_TILE = 512


def _vq_step(z_ref, z2_ref, m_ref, cb_ref, c2_ref, cbhi_ref, cblo_ref,
             kiota_ref, q_ref, idx_ref, sumsq_ref, cnt_ref):
    i = pl.program_id(0)
    z = z_ref[0]
    k = cb_ref.shape[0]

    dots = jax.lax.dot_general(
        z, cb_ref[...], (((1,), (1,)), ((), ())),
        preferred_element_type=jnp.float32)
    dist = (z2_ref[0] - 2.0 * dots) + c2_ref[...]

    minv = jnp.min(dist, axis=1, keepdims=True)
    kiota = kiota_ref[...]
    idx = jnp.min(jnp.where(dist == minv, kiota, k),
                  axis=1, keepdims=True)
    idx_ref[0] = idx

    onehot = (kiota == idx).astype(jnp.bfloat16)
    q_hi = jnp.dot(onehot, cbhi_ref[...],
                   preferred_element_type=jnp.float32)
    q_lo = jnp.dot(onehot, cblo_ref[...],
                   preferred_element_type=jnp.float32)
    q = q_hi + q_lo
    q_ref[0] = q

    m = m_ref[0]
    diff = z - q
    psum = jnp.sum(diff * diff * m, keepdims=True)
    pcnt = jnp.sum(m, keepdims=True)

    @pl.when(i == 0)
    def _init():
        sumsq_ref[...] = jnp.zeros((1, 1), jnp.float32)
        cnt_ref[...] = jnp.zeros((1, 1), jnp.float32)

    sumsq_ref[...] += psum
    cnt_ref[...] += pcnt


def kernel(z, mask, codebook):
    b, t, d = z.shape
    k = codebook.shape[0]
    n = b * t
    nt = n // _TILE

    z3 = z.reshape(nt, _TILE, d)
    z2 = jnp.sum(z * z, axis=-1).reshape(nt, _TILE, 1)
    m3 = mask.astype(jnp.float32).reshape(nt, _TILE, 1)
    c2 = jnp.sum(codebook * codebook, axis=-1).reshape(1, k)
    cb_hi = codebook.astype(jnp.bfloat16)
    cb_lo = (codebook - cb_hi.astype(jnp.float32)).astype(jnp.bfloat16)
    kiota = jax.lax.broadcasted_iota(jnp.int32, (_TILE, k), 1)

    grid = (nt,)
    q3, idx3, sumsq, cnt = pl.pallas_call(
        _vq_step,
        grid=grid,
        in_specs=[
            pl.BlockSpec((1, _TILE, d), lambda i: (i, 0, 0)),
            pl.BlockSpec((1, _TILE, 1), lambda i: (i, 0, 0)),
            pl.BlockSpec((1, _TILE, 1), lambda i: (i, 0, 0)),
            pl.BlockSpec((k, d), lambda i: (0, 0)),
            pl.BlockSpec((1, k), lambda i: (0, 0)),
            pl.BlockSpec((k, d), lambda i: (0, 0)),
            pl.BlockSpec((k, d), lambda i: (0, 0)),
            pl.BlockSpec((_TILE, k), lambda i: (0, 0)),
        ],
        out_specs=[
            pl.BlockSpec((1, _TILE, d), lambda i: (i, 0, 0)),
            pl.BlockSpec((1, _TILE, 1), lambda i: (i, 0, 0)),
            pl.BlockSpec((1, 1), lambda i: (0, 0)),
            pl.BlockSpec((1, 1), lambda i: (0, 0)),
        ],
        out_shape=[
            jax.ShapeDtypeStruct((nt, _TILE, d), jnp.float32),
            jax.ShapeDtypeStruct((nt, _TILE, 1), jnp.int32),
            jax.ShapeDtypeStruct((1, 1), jnp.float32),
            jax.ShapeDtypeStruct((1, 1), jnp.float32),
        ],
        compiler_params=pltpu.CompilerParams(
            dimension_semantics=("arbitrary",),
        ),
    )(z3, z2, m3, codebook, c2, cb_hi, cb_lo, kiota)

    quantized = q3.reshape(b, t, d)
    indices = idx3.reshape(b, t)
    denom = jnp.maximum(cnt[0, 0], 1.0) * jnp.float32(d)
    sum_commit_loss = sumsq[0, 0] / denom
    return quantized, indices, sum_commit_loss


# speedup vs baseline: 1.4972x; 1.0850x over previous
import jax
import jax.numpy as jnp
from jax.experimental import pallas as pl
from jax.experimental.pallas import tpu as pltpu

_TILE = 1024


def _vq_step(z_ref, z2_ref, m_ref, cb_ref, c2_ref, cbhi_ref, cblo_ref,
             kiota_ref, q_ref, idx_ref, sumsq_ref, cnt_ref):
    i = pl.program_id(0)
    z = z_ref[0]
    k = cb_ref.shape[0]

    dots = jax.lax.dot_general(
        z, cb_ref[...], (((1,), (1,)), ((), ())),
        preferred_element_type=jnp.float32)
    dist = (z2_ref[0] - 2.0 * dots) + c2_ref[...]

    minv = jnp.min(dist, axis=1, keepdims=True)
    kiota = kiota_ref[...]
    idx = jnp.min(jnp.where(dist == minv, kiota, k),
                  axis=1, keepdims=True)
    idx_ref[0] = idx

    onehot = (kiota == idx).astype(jnp.float32)
    q_hi = jnp.dot(onehot, cbhi_ref[...],
                   preferred_element_type=jnp.float32)
    q_lo = jnp.dot(onehot, cblo_ref[...],
                   preferred_element_type=jnp.float32)
    q = q_hi + q_lo
    q_ref[0] = q

    m = m_ref[0]
    diff = z - q
    psum = jnp.sum(diff * diff * m, keepdims=True)
    pcnt = jnp.sum(m, keepdims=True)

    @pl.when(i == 0)
    def _init():
        sumsq_ref[...] = jnp.zeros((1, 1), jnp.float32)
        cnt_ref[...] = jnp.zeros((1, 1), jnp.float32)

    sumsq_ref[...] += psum
    cnt_ref[...] += pcnt


def kernel(z, mask, codebook):
    b, t, d = z.shape
    k = codebook.shape[0]
    n = b * t
    nt = n // _TILE

    z3 = z.reshape(nt, _TILE, d)
    z2 = jnp.sum(z * z, axis=-1).reshape(nt, _TILE, 1)
    m3 = mask.astype(jnp.float32).reshape(nt, _TILE, 1)
    c2 = jnp.sum(codebook * codebook, axis=-1).reshape(1, k)
    cb_hi = codebook.astype(jnp.bfloat16).astype(jnp.float32)
    cb_lo = codebook - cb_hi
    kiota = jax.lax.broadcasted_iota(jnp.int32, (_TILE, k), 1)

    grid = (nt,)
    q3, idx3, sumsq, cnt = pl.pallas_call(
        _vq_step,
        grid=grid,
        in_specs=[
            pl.BlockSpec((1, _TILE, d), lambda i: (i, 0, 0)),
            pl.BlockSpec((1, _TILE, 1), lambda i: (i, 0, 0)),
            pl.BlockSpec((1, _TILE, 1), lambda i: (i, 0, 0)),
            pl.BlockSpec((k, d), lambda i: (0, 0)),
            pl.BlockSpec((1, k), lambda i: (0, 0)),
            pl.BlockSpec((k, d), lambda i: (0, 0)),
            pl.BlockSpec((k, d), lambda i: (0, 0)),
            pl.BlockSpec((_TILE, k), lambda i: (0, 0)),
        ],
        out_specs=[
            pl.BlockSpec((1, _TILE, d), lambda i: (i, 0, 0)),
            pl.BlockSpec((1, _TILE, 1), lambda i: (i, 0, 0)),
            pl.BlockSpec((1, 1), lambda i: (0, 0)),
            pl.BlockSpec((1, 1), lambda i: (0, 0)),
        ],
        out_shape=[
            jax.ShapeDtypeStruct((nt, _TILE, d), jnp.float32),
            jax.ShapeDtypeStruct((nt, _TILE, 1), jnp.int32),
            jax.ShapeDtypeStruct((1, 1), jnp.float32),
            jax.ShapeDtypeStruct((1, 1), jnp.float32),
        ],
        compiler_params=pltpu.CompilerParams(
            dimension_semantics=("arbitrary",),
        ),
    )(z3, z2, m3, codebook, c2, cb_hi, cb_lo, kiota)

    quantized = q3.reshape(b, t, d)
    indices = idx3.reshape(b, t)
    denom = jnp.maximum(cnt[0, 0], 1.0) * jnp.float32(d)
    sum_commit_loss = sumsq[0, 0] / denom
    return quantized, indices, sum_commit_loss


# speedup vs baseline: 2.1885x; 1.4617x over previous
import jax
import jax.numpy as jnp
from jax.experimental import pallas as pl
from jax.experimental.pallas import tpu as pltpu

_TILE = 1024


def _vq_step(z_ref, z2_ref, m_ref, cb_ref, c2_ref, cbhi_ref, cblo_ref,
             kiota_ref, q_ref, idx_ref, sumsq_ref, cnt_ref):
    i = pl.program_id(0)
    z = z_ref[0]
    k = cb_ref.shape[0]

    dots = jax.lax.dot_general(
        cb_ref[...], z, (((1,), (1,)), ((), ())),
        preferred_element_type=jnp.float32)
    dist = (z2_ref[0] - 2.0 * dots) + c2_ref[...]

    minv = jnp.min(dist, axis=0, keepdims=True)
    kiota = kiota_ref[...]
    idx = jnp.min(jnp.where(dist == minv, kiota, k),
                  axis=0, keepdims=True)
    idx_ref[0] = idx

    onehot = (kiota == idx).astype(jnp.float32)
    q_hi = jax.lax.dot_general(
        onehot, cbhi_ref[...], (((0,), (0,)), ((), ())),
        preferred_element_type=jnp.float32)
    q_lo = jax.lax.dot_general(
        onehot, cblo_ref[...], (((0,), (0,)), ((), ())),
        preferred_element_type=jnp.float32)
    q_ref[0] = q_hi + q_lo

    m = m_ref[0]
    psum = jnp.sum(minv * m, keepdims=True)
    pcnt = jnp.sum(m, keepdims=True)

    @pl.when(i == 0)
    def _init():
        sumsq_ref[...] = jnp.zeros((1, 1), jnp.float32)
        cnt_ref[...] = jnp.zeros((1, 1), jnp.float32)

    sumsq_ref[...] += psum
    cnt_ref[...] += pcnt


def kernel(z, mask, codebook):
    b, t, d = z.shape
    k = codebook.shape[0]
    n = b * t
    nt = n // _TILE

    z3 = z.reshape(nt, _TILE, d)
    z2 = jnp.sum(z * z, axis=-1).reshape(nt, 1, _TILE)
    m3 = mask.astype(jnp.float32).reshape(nt, 1, _TILE)
    c2 = jnp.sum(codebook * codebook, axis=-1).reshape(k, 1)
    cb_bits = jax.lax.bitcast_convert_type(codebook, jnp.uint32)
    cb_hi = jax.lax.bitcast_convert_type(
        cb_bits & jnp.uint32(0xFFFF0000), jnp.float32)
    cb_lo = codebook - cb_hi
    kiota = jax.lax.broadcasted_iota(jnp.int32, (k, _TILE), 0)

    grid = (nt,)
    q3, idx3, sumsq, cnt = pl.pallas_call(
        _vq_step,
        grid=grid,
        in_specs=[
            pl.BlockSpec((1, _TILE, d), lambda i: (i, 0, 0)),
            pl.BlockSpec((1, 1, _TILE), lambda i: (i, 0, 0)),
            pl.BlockSpec((1, 1, _TILE), lambda i: (i, 0, 0)),
            pl.BlockSpec((k, d), lambda i: (0, 0)),
            pl.BlockSpec((k, 1), lambda i: (0, 0)),
            pl.BlockSpec((k, d), lambda i: (0, 0)),
            pl.BlockSpec((k, d), lambda i: (0, 0)),
            pl.BlockSpec((k, _TILE), lambda i: (0, 0)),
        ],
        out_specs=[
            pl.BlockSpec((1, _TILE, d), lambda i: (i, 0, 0)),
            pl.BlockSpec((1, 1, _TILE), lambda i: (i, 0, 0)),
            pl.BlockSpec((1, 1), lambda i: (0, 0)),
            pl.BlockSpec((1, 1), lambda i: (0, 0)),
        ],
        out_shape=[
            jax.ShapeDtypeStruct((nt, _TILE, d), jnp.float32),
            jax.ShapeDtypeStruct((nt, 1, _TILE), jnp.int32),
            jax.ShapeDtypeStruct((1, 1), jnp.float32),
            jax.ShapeDtypeStruct((1, 1), jnp.float32),
        ],
        compiler_params=pltpu.CompilerParams(
            dimension_semantics=("arbitrary",),
        ),
    )(z3, z2, m3, codebook, c2, cb_hi, cb_lo, kiota)

    quantized = q3.reshape(b, t, d)
    indices = idx3.reshape(b, t)
    denom = jnp.maximum(cnt[0, 0], 1.0) * jnp.float32(d)
    sum_commit_loss = sumsq[0, 0] / denom
    return quantized, indices, sum_commit_loss


# speedup vs baseline: 2.3729x; 1.0843x over previous
import jax
import jax.numpy as jnp
from jax.experimental import pallas as pl
from jax.experimental.pallas import tpu as pltpu

_TILE = 1024


def _vq_step(z_ref, z2_ref, m_ref, cb_ref, c2_ref, cbhi_ref, cblo_ref,
             kiota_ref, q_ref, idx_ref, sumsq_ref, cnt_ref):
    i = pl.program_id(0)
    z = z_ref[0]
    k = cb_ref.shape[0]

    dots = jax.lax.dot_general(
        cb_ref[...], z, (((1,), (1,)), ((), ())),
        preferred_element_type=jnp.float32)
    dist = (z2_ref[0] - 2.0 * dots) + c2_ref[...]

    minv = jnp.min(dist, axis=0, keepdims=True)
    kiota = jnp.broadcast_to(kiota_ref[...], dist.shape)
    idx = jnp.min(jnp.where(dist == minv, kiota, k),
                  axis=0, keepdims=True)
    idx_ref[0] = idx

    onehot = (kiota == idx).astype(jnp.bfloat16)
    q_hi = jax.lax.dot_general(
        onehot, cbhi_ref[...], (((0,), (0,)), ((), ())),
        preferred_element_type=jnp.float32)
    q_lo = jax.lax.dot_general(
        onehot, cblo_ref[...], (((0,), (0,)), ((), ())),
        preferred_element_type=jnp.float32)
    q_ref[0] = q_hi + q_lo

    m = m_ref[0]
    psum = jnp.sum(minv * m, keepdims=True)
    pcnt = jnp.sum(m, keepdims=True)

    @pl.when(i == 0)
    def _init():
        sumsq_ref[...] = jnp.zeros((1, 1), jnp.float32)
        cnt_ref[...] = jnp.zeros((1, 1), jnp.float32)

    sumsq_ref[...] += psum
    cnt_ref[...] += pcnt


def kernel(z, mask, codebook):
    b, t, d = z.shape
    k = codebook.shape[0]
    n = b * t
    nt = n // _TILE

    z3 = z.reshape(nt, _TILE, d)
    z2 = jnp.sum(z * z, axis=-1).reshape(nt, 1, _TILE)
    m3 = mask.astype(jnp.float32).reshape(nt, 1, _TILE)
    c2 = jnp.sum(codebook * codebook, axis=-1).reshape(k, 1)
    cb_bits = jax.lax.bitcast_convert_type(codebook, jnp.uint32)
    cb_hi = jax.lax.bitcast_convert_type(
        cb_bits & jnp.uint32(0xFFFF0000), jnp.float32)
    cb_lo = codebook - cb_hi
    cb_hi = cb_hi.astype(jnp.bfloat16)
    cb_lo = cb_lo.astype(jnp.bfloat16)
    kiota = jax.lax.broadcasted_iota(jnp.int32, (k, 1), 0)

    grid = (nt,)
    q3, idx3, sumsq, cnt = pl.pallas_call(
        _vq_step,
        grid=grid,
        in_specs=[
            pl.BlockSpec((1, _TILE, d), lambda i: (i, 0, 0)),
            pl.BlockSpec((1, 1, _TILE), lambda i: (i, 0, 0)),
            pl.BlockSpec((1, 1, _TILE), lambda i: (i, 0, 0)),
            pl.BlockSpec((k, d), lambda i: (0, 0)),
            pl.BlockSpec((k, 1), lambda i: (0, 0)),
            pl.BlockSpec((k, d), lambda i: (0, 0)),
            pl.BlockSpec((k, d), lambda i: (0, 0)),
            pl.BlockSpec((k, 1), lambda i: (0, 0)),
        ],
        out_specs=[
            pl.BlockSpec((1, _TILE, d), lambda i: (i, 0, 0)),
            pl.BlockSpec((1, 1, _TILE), lambda i: (i, 0, 0)),
            pl.BlockSpec((1, 1), lambda i: (0, 0)),
            pl.BlockSpec((1, 1), lambda i: (0, 0)),
        ],
        out_shape=[
            jax.ShapeDtypeStruct((nt, _TILE, d), jnp.float32),
            jax.ShapeDtypeStruct((nt, 1, _TILE), jnp.int32),
            jax.ShapeDtypeStruct((1, 1), jnp.float32),
            jax.ShapeDtypeStruct((1, 1), jnp.float32),
        ],
        compiler_params=pltpu.CompilerParams(
            dimension_semantics=("arbitrary",),
        ),
    )(z3, z2, m3, codebook, c2, cb_hi, cb_lo, kiota)

    quantized = q3.reshape(b, t, d)
    indices = idx3.reshape(b, t)
    denom = jnp.maximum(cnt[0, 0], 1.0) * jnp.float32(d)
    sum_commit_loss = sumsq[0, 0] / denom
    return quantized, indices, sum_commit_loss


# speedup vs baseline: 2.5751x; 1.0852x over previous
import functools

import jax
import jax.numpy as jnp
from jax import lax
from jax.experimental import pallas as pl
from jax.experimental.pallas import tpu as pltpu
from jax.experimental.pallas import tpu_sc as plsc

_TILE = 1024
_CHUNK = 128


def _vq_step(z_ref, z2_ref, m_ref, cb_ref, c2_ref, kiota_ref,
             idx_ref, sumsq_ref, cnt_ref):
    i = pl.program_id(0)
    z = z_ref[0]
    k = cb_ref.shape[0]

    dots = jax.lax.dot_general(
        cb_ref[...], z, (((1,), (1,)), ((), ())),
        preferred_element_type=jnp.float32)
    dist = (z2_ref[0] - 2.0 * dots) + c2_ref[...]

    minv = jnp.min(dist, axis=0, keepdims=True)
    kiota = jnp.broadcast_to(kiota_ref[...], dist.shape)
    idx = jnp.min(jnp.where(dist == minv, kiota, k),
                  axis=0, keepdims=True)
    idx_ref[0] = idx

    m = m_ref[0]
    psum = jnp.sum(minv * m, keepdims=True)
    pcnt = jnp.sum(m, keepdims=True)

    @pl.when(i == 0)
    def _init():
        sumsq_ref[...] = jnp.zeros((1, 1), jnp.float32)
        cnt_ref[...] = jnp.zeros((1, 1), jnp.float32)

    sumsq_ref[...] += psum
    cnt_ref[...] += pcnt


def _tc_indices_loss(z, mask, codebook):
    b, t, d = z.shape
    k = codebook.shape[0]
    n = b * t
    nt = n // _TILE

    z3 = z.reshape(nt, _TILE, d)
    z2 = jnp.sum(z * z, axis=-1).reshape(nt, 1, _TILE)
    m3 = mask.astype(jnp.float32).reshape(nt, 1, _TILE)
    c2 = jnp.sum(codebook * codebook, axis=-1).reshape(k, 1)
    kiota = jax.lax.broadcasted_iota(jnp.int32, (k, 1), 0)

    idx3, sumsq, cnt = pl.pallas_call(
        _vq_step,
        grid=(nt,),
        in_specs=[
            pl.BlockSpec((1, _TILE, d), lambda i: (i, 0, 0)),
            pl.BlockSpec((1, 1, _TILE), lambda i: (i, 0, 0)),
            pl.BlockSpec((1, 1, _TILE), lambda i: (i, 0, 0)),
            pl.BlockSpec((k, d), lambda i: (0, 0)),
            pl.BlockSpec((k, 1), lambda i: (0, 0)),
            pl.BlockSpec((k, 1), lambda i: (0, 0)),
        ],
        out_specs=[
            pl.BlockSpec((1, 1, _TILE), lambda i: (i, 0, 0)),
            pl.BlockSpec((1, 1), lambda i: (0, 0)),
            pl.BlockSpec((1, 1), lambda i: (0, 0)),
        ],
        out_shape=[
            jax.ShapeDtypeStruct((nt, 1, _TILE), jnp.int32),
            jax.ShapeDtypeStruct((1, 1), jnp.float32),
            jax.ShapeDtypeStruct((1, 1), jnp.float32),
        ],
        compiler_params=pltpu.CompilerParams(
            dimension_semantics=("arbitrary",),
        ),
    )(z3, z2, m3, codebook, c2, kiota)
    return idx3.reshape(b, t), sumsq[0, 0], cnt[0, 0]


def _sc_gather(codebook, indices):
    nrows, d = codebook.shape[0], codebook.shape[1]
    nidx = indices.shape[0]
    info = plsc.get_sparse_core_info()
    nw = info.num_cores * info.num_subcores
    b_per_w = nidx // nw
    nchunks = b_per_w // _CHUNK
    mesh = plsc.VectorSubcoreMesh(core_axis_name="c", subcore_axis_name="s")

    @functools.partial(
        pl.kernel, mesh=mesh,
        out_type=jax.ShapeDtypeStruct((nidx, d), jnp.float32),
        scratch_types=[
            pltpu.VMEM((_CHUNK,), jnp.int32),
            pltpu.VMEM((_CHUNK, d), jnp.float32),
            pltpu.SemaphoreType.DMA,
        ],
    )
    def gather_k(cb_hbm, idx_hbm, out_hbm, idx_v, rows_v, sem):
        wid = lax.axis_index("s") * info.num_cores + lax.axis_index("c")
        base = wid * b_per_w
        for j in range(nchunks):
            off = base + j * _CHUNK
            pltpu.sync_copy(idx_hbm.at[pl.ds(off, _CHUNK)], idx_v)
            pltpu.async_copy(cb_hbm.at[idx_v], rows_v, sem).wait()
            pltpu.sync_copy(rows_v, out_hbm.at[pl.ds(off, _CHUNK)])

    return gather_k(codebook, indices)


def kernel(z, mask, codebook):
    b, t, d = z.shape
    indices, sumsq, cnt = _tc_indices_loss(z, mask, codebook)
    rows = _sc_gather(codebook, indices.reshape(b * t))
    quantized = rows.reshape(b, t, d)
    denom = jnp.maximum(cnt, 1.0) * jnp.float32(d)
    sum_commit_loss = sumsq / denom
    return quantized, indices, sum_commit_loss
